# baseline (speedup 1.0000x reference)
import functools

import numpy as np
import jax
import jax.numpy as jnp
from jax import lax
from jax.experimental import pallas as pl
from jax.experimental.pallas import tpu as pltpu
from jax.experimental.pallas import tpu_sc as plsc

DIM = 1024
INTER = 1024
NE = 15
TOPK = 7
SEQ = 2048
LANES = 128
BLK = 256
NASSIGN = SEQ * TOPK
PAD_TOTAL = NASSIGN + NE * BLK
NBLOCKS = PAD_TOTAL // BLK
NW = 32
APW = NASSIGN // NW
TPW = SEQ // NW
_NCH = 8
_CH = APW // _NCH
_TS = _CH // TOPK
GW = 128

_NEG = -1e30

_LSTRICT = np.tril(np.ones((256, 256), np.float32), k=-1).astype(jnp.bfloat16)
_TOKFLAT = (np.arange(NASSIGN, dtype=np.int32) // TOPK).reshape(NW, _NCH, _CH)


def _gelu_exact(h):
    return 0.5 * h * (1.0 + jax.lax.erf(h * 0.7071067811865476))


def _k1s_body(x_ref, xb_ref, w1s_ref, b1s_ref, w2s_ref, b2s_ref, y0_ref):
    h = jnp.dot(xb_ref[...], w1s_ref[...], preferred_element_type=jnp.float32)
    h = _gelu_exact(h + b1s_ref[...])
    sh = jnp.dot(h.astype(jnp.bfloat16), w2s_ref[...],
                 preferred_element_type=jnp.float32) + b2s_ref[...]
    y0_ref[...] = x_ref[...] + sh


def _k1r_body(x_ref, wr_ref, brp_ref, l_ref,
              tkv_ref, tke_ref, tkr_ref, cnt_ref):
    i = pl.program_id(0)

    @pl.when(i == 0)
    def _():
        cnt_ref[...] = jnp.zeros_like(cnt_ref)

    logits = jnp.dot(x_ref[...], wr_ref[...],
                     preferred_element_type=jnp.float32) + brp_ref[...]
    m = jnp.max(logits, axis=-1, keepdims=True)
    ex = jnp.exp(logits - m)
    aff = ex / jnp.sum(ex, axis=-1, keepdims=True)

    blk = aff.shape[0]
    lane = jax.lax.broadcasted_iota(jnp.int32, (blk, LANES), 1)

    def _argmax_rounds(fn):
        work = aff
        for k in range(TOPK):
            mx = jnp.max(work, axis=-1, keepdims=True)
            ismx = work == mx
            first = jnp.min(jnp.where(ismx, lane, jnp.int32(1 << 30)),
                            axis=-1, keepdims=True)
            chosen = lane == first
            fn(k, chosen)
            work = jnp.where(chosen, _NEG, work)

    mask_acc = [jnp.zeros((blk, LANES), jnp.float32)]

    def _collect_mask(_, chosen):
        mask_acc[0] = mask_acc[0] + jnp.where(chosen, 1.0, 0.0)

    _argmax_rounds(_collect_mask)
    memb = mask_acc[0]

    cumex = jnp.dot(l_ref[...], memb.astype(jnp.bfloat16),
                    preferred_element_type=jnp.float32)
    rank_all = cnt_ref[...] + cumex

    cols = [jnp.zeros((blk, LANES), jnp.float32) for _ in range(3)]
    lanef = lane.astype(jnp.float32)

    def _collect_cols(k, chosen):
        kcol = (lane == k).astype(jnp.float32)
        qv = jnp.sum(jnp.where(chosen, aff, 0.0), axis=-1, keepdims=True)
        qe = jnp.sum(jnp.where(chosen, lanef, 0.0), axis=-1, keepdims=True)
        qr = jnp.sum(jnp.where(chosen, rank_all, 0.0), axis=-1, keepdims=True)
        cols[0] = cols[0] + kcol * qv
        cols[1] = cols[1] + kcol * qe
        cols[2] = cols[2] + kcol * qr

    _argmax_rounds(_collect_cols)
    tkv_ref[...] = cols[0]
    tke_ref[...] = cols[1].astype(jnp.int32)
    tkr_ref[...] = cols[2].astype(jnp.int32)
    cnt_ref[...] += jnp.sum(memb, axis=0, keepdims=True)


def _kpos_body(tke_ref, tkr_ref, tkv_ref, st_ref, pos_ref, v16_ref):
    lane1 = jax.lax.broadcasted_iota(jnp.int32, (1, LANES), 1)
    lane = jax.lax.broadcasted_iota(jnp.int32, (SEQ, LANES), 1)
    tke = tke_ref[...]
    tkv = tkv_ref[...]
    pos = tkr_ref[...]
    st = st_ref[...]
    for e in range(NE):
        se = jnp.sum(jnp.where(lane1 == e, st, 0), axis=-1, keepdims=True)
        pos = pos + jnp.where(tke == e, se, 0)
    pos_ref[...] = pos
    lanew = jax.lax.broadcasted_iota(jnp.int32, (SEQ, TOPK * GW), 1)
    v16 = jnp.zeros((SEQ, TOPK * GW), jnp.float32)
    for k in range(TOPK):
        vk = jnp.sum(jnp.where(lane == k, tkv, 0.0), axis=-1, keepdims=True)
        v16 = v16 + jnp.where((lanew // GW) == k, vk, 0.0)
    v16_ref[...] = v16


def _k4_body(be_ref, disp_ref, w1_ref, b1_ref, w2_ref, b2_ref, val_ref,
             out_ref):
    del be_ref
    xb = disp_ref[...].astype(jnp.bfloat16)
    h = jnp.dot(xb, w1_ref[0], preferred_element_type=jnp.float32)
    h = _gelu_exact(h + b1_ref[0])
    eo = jnp.dot(h.astype(jnp.bfloat16), w2_ref[0],
                 preferred_element_type=jnp.float32) + b2_ref[0]
    out_ref[...] = eo * val_ref[:, 0:1]


@functools.cache
def _build_sc_dispatch():
    mesh = plsc.VectorSubcoreMesh(core_axis_name="c", subcore_axis_name="s")

    @functools.partial(
        pl.kernel,
        mesh=mesh,
        out_type=[
            jax.ShapeDtypeStruct((PAD_TOTAL, DIM), jnp.float32),
            jax.ShapeDtypeStruct((PAD_TOTAL, GW), jnp.float32),
        ],
        scratch_types=[
            pltpu.VMEM((_NCH, _CH), jnp.int32),
            pltpu.VMEM((_NCH, _CH), jnp.int32),
            pltpu.VMEM((_CH, GW), jnp.float32),
            pltpu.VMEM((2, _CH, DIM), jnp.float32),
            pltpu.SemaphoreType.DMA,
            pltpu.SemaphoreType.DMA,
            pltpu.SemaphoreType.DMA,
            pltpu.SemaphoreType.DMA,
            pltpu.SemaphoreType.DMA,
            pltpu.SemaphoreType.DMA,
        ],
    )
    def _dispatch(xb_hbm, tok_hbm, pos_hbm, v16_hbm, disp_hbm, vdisp_hbm,
                  tokv, posv, valv, rows, sg0, sg1, ss0, ss1, sv0, sv1):
        wid = lax.axis_index("s") * 2 + lax.axis_index("c")
        pltpu.sync_copy(tok_hbm.at[wid], tokv)
        pltpu.sync_copy(pos_hbm.at[wid], posv)
        sgs, sss = [sg0, sg1], [ss0, ss1]
        gd, sd, vd = [None, None], [None, None], [None]
        del sv1

        def start_gather(c):
            gd[c % 2] = pltpu.async_copy(xb_hbm.at[tokv.at[c]],
                                         rows.at[c % 2], sgs[c % 2])

        start_gather(0)
        for c in range(_NCH):
            gd[c % 2].wait()
            sd[c % 2] = pltpu.async_copy(rows.at[c % 2],
                                         disp_hbm.at[posv.at[c]], sss[c % 2])
            if c >= 1:
                vd[0].wait()
            pltpu.sync_copy(v16_hbm.at[wid, c], valv)
            vd[0] = pltpu.async_copy(valv, vdisp_hbm.at[posv.at[c]], sv0)
            if c + 1 < _NCH:
                if c >= 1:
                    sd[(c + 1) % 2].wait()
                start_gather(c + 1)
        sd[0].wait()
        sd[1].wait()
        vd[0].wait()

    return _dispatch


def _sc_dispatch(xb, tokflat, pos_r, v16_r):
    return _build_sc_dispatch()(xb, tokflat, pos_r, v16_r)


@functools.cache
def _build_sc_combine():
    mesh = plsc.VectorSubcoreMesh(core_axis_name="c", subcore_axis_name="s")

    @functools.partial(
        pl.kernel,
        mesh=mesh,
        out_type=jax.ShapeDtypeStruct((SEQ // _TS, _TS, DIM), jnp.float32),
        scratch_types=[
            pltpu.VMEM((_NCH, _CH), jnp.int32),
            pltpu.VMEM((2, _CH, DIM), jnp.float32),
            pltpu.VMEM((_TS, DIM), jnp.float32),
            pltpu.SemaphoreType.DMA,
            pltpu.SemaphoreType.DMA,
        ],
    )
    def _combine(eo_hbm, y0_hbm, pos_hbm, out_hbm,
                 posv, rows, y0b, sg0, sg1):
        wid = lax.axis_index("s") * 2 + lax.axis_index("c")
        pltpu.sync_copy(pos_hbm.at[wid], posv)
        sgs = [sg0, sg1]
        gd = [None, None]

        def start_gather(c):
            gd[c % 2] = pltpu.async_copy(eo_hbm.at[posv.at[c]],
                                         rows.at[c % 2], sgs[c % 2])

        start_gather(0)
        start_gather(1)
        for c in range(_NCH):
            gd[c % 2].wait()
            pltpu.sync_copy(y0_hbm.at[wid * _NCH + c], y0b)

            def _lanes(i, _):
                for t in range(_TS):
                    acc = y0b[t, pl.ds(i * 16, 16)]
                    for k in range(TOPK):
                        acc = acc + rows[c % 2, t * TOPK + k,
                                         pl.ds(i * 16, 16)]
                    y0b[t, pl.ds(i * 16, 16)] = acc
                return 0

            lax.fori_loop(0, DIM // 16, _lanes, 0)
            pltpu.sync_copy(y0b, out_hbm.at[wid * _NCH + c])
            if c + 2 < _NCH:
                start_gather(c + 2)

    return _combine


def _sc_combine(eo, y0r, pos_r):
    return _build_sc_combine()(eo, y0r, pos_r)


def kernel(x, W1s, b1s, W2s, b2s, W1r, b1r, W2r, b2r, Wr, br):
    x2 = x.reshape(SEQ, DIM)
    xb = x2.astype(jnp.bfloat16)
    w1s = W1s.astype(jnp.bfloat16)
    w2s = W2s.astype(jnp.bfloat16)
    w1r = W1r.astype(jnp.bfloat16)
    w2r = W2r.astype(jnp.bfloat16)
    wr_p = jnp.pad(Wr, ((0, 0), (0, LANES - NE)))
    br_p = jnp.pad(br, (0, LANES - NE), constant_values=_NEG).reshape(1, LANES)
    b1s2 = b1s.reshape(1, INTER)
    b2s2 = b2s.reshape(1, DIM)
    lstrict = jnp.asarray(_LSTRICT)
    tokflat = jnp.asarray(_TOKFLAT)

    B1 = 256
    tkv, tke, tkr, cnt = pl.pallas_call(
        _k1r_body,
        grid=(SEQ // B1,),
        in_specs=[
            pl.BlockSpec((B1, DIM), lambda i: (i, 0)),
            pl.BlockSpec((DIM, LANES), lambda i: (0, 0)),
            pl.BlockSpec((1, LANES), lambda i: (0, 0)),
            pl.BlockSpec((B1, B1), lambda i: (0, 0)),
        ],
        out_specs=[
            pl.BlockSpec((B1, LANES), lambda i: (i, 0)),
            pl.BlockSpec((B1, LANES), lambda i: (i, 0)),
            pl.BlockSpec((B1, LANES), lambda i: (i, 0)),
            pl.BlockSpec((1, LANES), lambda i: (0, 0)),
        ],
        out_shape=[
            jax.ShapeDtypeStruct((SEQ, LANES), jnp.float32),
            jax.ShapeDtypeStruct((SEQ, LANES), jnp.int32),
            jax.ShapeDtypeStruct((SEQ, LANES), jnp.int32),
            jax.ShapeDtypeStruct((1, LANES), jnp.float32),
        ],
    )(x2, wr_p, br_p, lstrict)

    y0 = pl.pallas_call(
        _k1s_body,
        grid=(SEQ // B1,),
        in_specs=[
            pl.BlockSpec((B1, DIM), lambda i: (i, 0)),
            pl.BlockSpec((B1, DIM), lambda i: (i, 0)),
            pl.BlockSpec((DIM, INTER), lambda i: (0, 0)),
            pl.BlockSpec((1, INTER), lambda i: (0, 0)),
            pl.BlockSpec((INTER, DIM), lambda i: (0, 0)),
            pl.BlockSpec((1, DIM), lambda i: (0, 0)),
        ],
        out_specs=pl.BlockSpec((B1, DIM), lambda i: (i, 0)),
        out_shape=jax.ShapeDtypeStruct((SEQ, DIM), jnp.float32),
    )(x2, xb, w1s, b1s2, w2s, b2s2)

    counts = cnt[0, :NE].astype(jnp.int32)
    padded = ((counts + BLK - 1) // BLK) * BLK
    starts = jnp.concatenate(
        [jnp.zeros((1,), jnp.int32), jnp.cumsum(padded)])[:NE]
    starts_p = jnp.pad(starts, (0, LANES - NE)).reshape(1, LANES)
    bidx = jnp.arange(NBLOCKS, dtype=jnp.int32) * BLK
    block_expert = jnp.sum((bidx[:, None] >= starts[None, :]).astype(jnp.int32),
                           axis=1) - 1

    pos, v16 = pl.pallas_call(
        _kpos_body,
        grid=(1,),
        in_specs=[
            pl.BlockSpec((SEQ, LANES), lambda i: (0, 0)),
            pl.BlockSpec((SEQ, LANES), lambda i: (0, 0)),
            pl.BlockSpec((SEQ, LANES), lambda i: (0, 0)),
            pl.BlockSpec((1, LANES), lambda i: (0, 0)),
        ],
        out_specs=[
            pl.BlockSpec((SEQ, LANES), lambda i: (0, 0)),
            pl.BlockSpec((SEQ, TOPK * GW), lambda i: (0, 0)),
        ],
        out_shape=[
            jax.ShapeDtypeStruct((SEQ, LANES), jnp.int32),
            jax.ShapeDtypeStruct((SEQ, TOPK * GW), jnp.float32),
        ],
    )(tke, tkr, tkv, starts_p)

    pos_r = pos[:, :TOPK].reshape(NW, _NCH, _CH)
    v16_r = v16.reshape(NW, _NCH, _CH, GW)

    disp, vdisp = _sc_dispatch(x2, tokflat, pos_r, v16_r)

    eo = pl.pallas_call(
        _k4_body,
        grid_spec=pltpu.PrefetchScalarGridSpec(
            num_scalar_prefetch=1,
            grid=(NBLOCKS,),
            in_specs=[
                pl.BlockSpec((BLK, DIM), lambda i, be: (i, 0)),
                pl.BlockSpec((1, DIM, INTER), lambda i, be: (be[i], 0, 0)),
                pl.BlockSpec((1, 1, INTER), lambda i, be: (be[i], 0, 0)),
                pl.BlockSpec((1, INTER, DIM), lambda i, be: (be[i], 0, 0)),
                pl.BlockSpec((1, 1, DIM), lambda i, be: (be[i], 0, 0)),
                pl.BlockSpec((BLK, GW), lambda i, be: (i, 0)),
            ],
            out_specs=pl.BlockSpec((BLK, DIM), lambda i, be: (i, 0)),
        ),
        out_shape=jax.ShapeDtypeStruct((PAD_TOTAL, DIM), jnp.float32),
    )(block_expert, disp, w1r, b1r.reshape(NE, 1, INTER), w2r,
      b2r.reshape(NE, 1, DIM), vdisp)

    out = _sc_combine(eo, y0.reshape(SEQ // _TS, _TS, DIM), pos_r)
    return out.reshape(1, SEQ, DIM)


# speedup vs baseline: 1.0800x; 1.0800x over previous
import functools

import numpy as np
import jax
import jax.numpy as jnp
from jax import lax
from jax.experimental import pallas as pl
from jax.experimental.pallas import tpu as pltpu
from jax.experimental.pallas import tpu_sc as plsc

DIM = 1024
INTER = 1024
NE = 15
TOPK = 7
SEQ = 2048
LANES = 128
BLK = 256
NASSIGN = SEQ * TOPK
PAD_TOTAL = NASSIGN + NE * BLK
NBLOCKS = PAD_TOTAL // BLK
NW = 32
APW = NASSIGN // NW
TPW = SEQ // NW
_NCH = 8
_CH = APW // _NCH
_TS = _CH // TOPK
GW = 128

_NEG = -1e30

_LSTRICT = np.tril(np.ones((256, 256), np.float32), k=-1).astype(jnp.bfloat16)
_TOKFLAT = (np.arange(NASSIGN, dtype=np.int32) // TOPK).reshape(NW, _NCH, _CH)


def _gelu_exact(h):
    return 0.5 * h * (1.0 + jax.lax.erf(h * 0.7071067811865476))


def _k1_body(x_ref, xb_ref, w1s_ref, b1s_ref, w2s_ref, b2s_ref,
             wr_ref, brp_ref, l_ref,
             y0_ref, tkv_ref, tke_ref, tkr_ref, cnt_ref):
    i = pl.program_id(0)

    @pl.when(i == 0)
    def _():
        cnt_ref[...] = jnp.zeros_like(cnt_ref)

    h = jnp.dot(xb_ref[...], w1s_ref[...], preferred_element_type=jnp.float32)
    h = _gelu_exact(h + b1s_ref[...])
    sh = jnp.dot(h.astype(jnp.bfloat16), w2s_ref[...],
                 preferred_element_type=jnp.float32) + b2s_ref[...]
    y0_ref[...] = x_ref[...] + sh

    logits = jnp.dot(x_ref[...], wr_ref[...],
                     preferred_element_type=jnp.float32) + brp_ref[...]
    m = jnp.max(logits, axis=-1, keepdims=True)
    ex = jnp.exp(logits - m)
    aff = ex / jnp.sum(ex, axis=-1, keepdims=True)

    blk = aff.shape[0]
    lane = jax.lax.broadcasted_iota(jnp.int32, (blk, LANES), 1)

    def _argmax_rounds(fn):
        work = aff
        for k in range(TOPK):
            mx = jnp.max(work, axis=-1, keepdims=True)
            ismx = work == mx
            first = jnp.min(jnp.where(ismx, lane, jnp.int32(1 << 30)),
                            axis=-1, keepdims=True)
            chosen = lane == first
            fn(k, chosen)
            work = jnp.where(chosen, _NEG, work)

    mask_acc = [jnp.zeros((blk, LANES), jnp.float32)]

    def _collect_mask(_, chosen):
        mask_acc[0] = mask_acc[0] + jnp.where(chosen, 1.0, 0.0)

    _argmax_rounds(_collect_mask)
    memb = mask_acc[0]

    cumex = jnp.dot(l_ref[...], memb.astype(jnp.bfloat16),
                    preferred_element_type=jnp.float32)
    rank_all = cnt_ref[...] + cumex

    cols = [jnp.zeros((blk, LANES), jnp.float32) for _ in range(3)]
    lanef = lane.astype(jnp.float32)

    def _collect_cols(k, chosen):
        kcol = (lane == k).astype(jnp.float32)
        qv = jnp.sum(jnp.where(chosen, aff, 0.0), axis=-1, keepdims=True)
        qe = jnp.sum(jnp.where(chosen, lanef, 0.0), axis=-1, keepdims=True)
        qr = jnp.sum(jnp.where(chosen, rank_all, 0.0), axis=-1, keepdims=True)
        cols[0] = cols[0] + kcol * qv
        cols[1] = cols[1] + kcol * qe
        cols[2] = cols[2] + kcol * qr

    _argmax_rounds(_collect_cols)
    tkv_ref[...] = cols[0]
    tke_ref[...] = cols[1].astype(jnp.int32)
    tkr_ref[...] = cols[2].astype(jnp.int32)
    cnt_ref[...] += jnp.sum(memb, axis=0, keepdims=True)


def _kpos_body(tke_ref, tkr_ref, tkv_ref, st_ref, pos_ref, v16_ref):
    lane1 = jax.lax.broadcasted_iota(jnp.int32, (1, LANES), 1)
    lane = jax.lax.broadcasted_iota(jnp.int32, (SEQ, LANES), 1)
    tke = tke_ref[...]
    tkv = tkv_ref[...]
    pos = tkr_ref[...]
    st = st_ref[...]
    for e in range(NE):
        se = jnp.sum(jnp.where(lane1 == e, st, 0), axis=-1, keepdims=True)
        pos = pos + jnp.where(tke == e, se, 0)
    pos_ref[...] = pos
    lanew = jax.lax.broadcasted_iota(jnp.int32, (SEQ, TOPK * GW), 1)
    v16 = jnp.zeros((SEQ, TOPK * GW), jnp.float32)
    for k in range(TOPK):
        vk = jnp.sum(jnp.where(lane == k, tkv, 0.0), axis=-1, keepdims=True)
        v16 = v16 + jnp.where((lanew // GW) == k, vk, 0.0)
    v16_ref[...] = v16


def _k4_body(be_ref, disp_ref, w1_ref, b1_ref, w2_ref, b2_ref, val_ref,
             out_ref):
    del be_ref
    xb = disp_ref[...].astype(jnp.bfloat16)
    h = jnp.dot(xb, w1_ref[0], preferred_element_type=jnp.float32)
    h = _gelu_exact(h + b1_ref[0])
    eo = jnp.dot(h.astype(jnp.bfloat16), w2_ref[0],
                 preferred_element_type=jnp.float32) + b2_ref[0]
    out_ref[...] = eo * val_ref[:, 0:1]


@functools.cache
def _build_sc_dispatch():
    mesh = plsc.VectorSubcoreMesh(core_axis_name="c", subcore_axis_name="s")

    @functools.partial(
        pl.kernel,
        mesh=mesh,
        out_type=[
            jax.ShapeDtypeStruct((PAD_TOTAL, DIM), jnp.float32),
            jax.ShapeDtypeStruct((PAD_TOTAL, GW), jnp.float32),
        ],
        scratch_types=[
            pltpu.VMEM((TPW, DIM), jnp.float32),
            pltpu.VMEM((TOPK, TPW), jnp.int32),
            pltpu.VMEM((2, TPW, GW), jnp.float32),
            pltpu.SemaphoreType.DMA,
            pltpu.SemaphoreType.DMA,
            pltpu.SemaphoreType.DMA,
        ],
    )
    def _dispatch(x3_hbm, posk_hbm, v16k_hbm, disp_hbm, vdisp_hbm,
                  xrows, posv, valv, ss0, sv0, sv1):
        wid = lax.axis_index("s") * 2 + lax.axis_index("c")
        pltpu.sync_copy(x3_hbm.at[wid], xrows)
        pltpu.sync_copy(posk_hbm.at[wid], posv)
        sd = []
        for k in range(TOPK):
            sd.append(pltpu.async_copy(xrows, disp_hbm.at[posv.at[k]], ss0))
        svs = [sv0, sv1]
        vd = [None, None]
        for k in range(TOPK):
            if k >= 2:
                vd[k % 2].wait()
            pltpu.sync_copy(v16k_hbm.at[wid, k], valv.at[k % 2])
            vd[k % 2] = pltpu.async_copy(valv.at[k % 2],
                                         vdisp_hbm.at[posv.at[k]],
                                         svs[k % 2])
        for d in sd:
            d.wait()
        vd[0].wait()
        vd[1].wait()

    return _dispatch


def _sc_dispatch(x3, posk, v16k):
    return _build_sc_dispatch()(x3, posk, v16k)


@functools.cache
def _build_sc_combine():
    mesh = plsc.VectorSubcoreMesh(core_axis_name="c", subcore_axis_name="s")

    @functools.partial(
        pl.kernel,
        mesh=mesh,
        out_type=jax.ShapeDtypeStruct((SEQ // _TS, _TS, DIM), jnp.float32),
        scratch_types=[
            pltpu.VMEM((_NCH, _CH), jnp.int32),
            pltpu.VMEM((2, _CH, DIM), jnp.float32),
            pltpu.VMEM((_TS, DIM), jnp.float32),
            pltpu.SemaphoreType.DMA,
            pltpu.SemaphoreType.DMA,
        ],
    )
    def _combine(eo_hbm, y0_hbm, pos_hbm, out_hbm,
                 posv, rows, y0b, sg0, sg1):
        wid = lax.axis_index("s") * 2 + lax.axis_index("c")
        pltpu.sync_copy(pos_hbm.at[wid], posv)
        sgs = [sg0, sg1]
        gd = [None, None]

        def start_gather(c):
            gd[c % 2] = pltpu.async_copy(eo_hbm.at[posv.at[c]],
                                         rows.at[c % 2], sgs[c % 2])

        start_gather(0)
        start_gather(1)
        for c in range(_NCH):
            gd[c % 2].wait()
            pltpu.sync_copy(y0_hbm.at[wid * _NCH + c], y0b)

            def _lanes(i, _):
                for t in range(_TS):
                    acc = y0b[t, pl.ds(i * 16, 16)]
                    for k in range(TOPK):
                        acc = acc + rows[c % 2, t * TOPK + k,
                                         pl.ds(i * 16, 16)]
                    y0b[t, pl.ds(i * 16, 16)] = acc
                return 0

            lax.fori_loop(0, DIM // 16, _lanes, 0)
            pltpu.sync_copy(y0b, out_hbm.at[wid * _NCH + c])
            if c + 2 < _NCH:
                start_gather(c + 2)

    return _combine


def _sc_combine(eo, y0r, pos_r):
    return _build_sc_combine()(eo, y0r, pos_r)


def kernel(x, W1s, b1s, W2s, b2s, W1r, b1r, W2r, b2r, Wr, br):
    x2 = x.reshape(SEQ, DIM)
    xb = x2.astype(jnp.bfloat16)
    w1s = W1s.astype(jnp.bfloat16)
    w2s = W2s.astype(jnp.bfloat16)
    w1r = W1r.astype(jnp.bfloat16)
    w2r = W2r.astype(jnp.bfloat16)
    wr_p = jnp.pad(Wr, ((0, 0), (0, LANES - NE)))
    br_p = jnp.pad(br, (0, LANES - NE), constant_values=_NEG).reshape(1, LANES)
    b1s2 = b1s.reshape(1, INTER)
    b2s2 = b2s.reshape(1, DIM)
    lstrict = jnp.asarray(_LSTRICT)

    B1 = 256
    y0, tkv, tke, tkr, cnt = pl.pallas_call(
        _k1_body,
        grid=(SEQ // B1,),
        in_specs=[
            pl.BlockSpec((B1, DIM), lambda i: (i, 0)),
            pl.BlockSpec((B1, DIM), lambda i: (i, 0)),
            pl.BlockSpec((DIM, INTER), lambda i: (0, 0)),
            pl.BlockSpec((1, INTER), lambda i: (0, 0)),
            pl.BlockSpec((INTER, DIM), lambda i: (0, 0)),
            pl.BlockSpec((1, DIM), lambda i: (0, 0)),
            pl.BlockSpec((DIM, LANES), lambda i: (0, 0)),
            pl.BlockSpec((1, LANES), lambda i: (0, 0)),
            pl.BlockSpec((B1, B1), lambda i: (0, 0)),
        ],
        out_specs=[
            pl.BlockSpec((B1, DIM), lambda i: (i, 0)),
            pl.BlockSpec((B1, LANES), lambda i: (i, 0)),
            pl.BlockSpec((B1, LANES), lambda i: (i, 0)),
            pl.BlockSpec((B1, LANES), lambda i: (i, 0)),
            pl.BlockSpec((1, LANES), lambda i: (0, 0)),
        ],
        out_shape=[
            jax.ShapeDtypeStruct((SEQ, DIM), jnp.float32),
            jax.ShapeDtypeStruct((SEQ, LANES), jnp.float32),
            jax.ShapeDtypeStruct((SEQ, LANES), jnp.int32),
            jax.ShapeDtypeStruct((SEQ, LANES), jnp.int32),
            jax.ShapeDtypeStruct((1, LANES), jnp.float32),
        ],
    )(x2, xb, w1s, b1s2, w2s, b2s2, wr_p, br_p, lstrict)

    counts = cnt[0, :NE].astype(jnp.int32)
    padded = ((counts + BLK - 1) // BLK) * BLK
    starts = jnp.concatenate(
        [jnp.zeros((1,), jnp.int32), jnp.cumsum(padded)])[:NE]
    starts_p = jnp.pad(starts, (0, LANES - NE)).reshape(1, LANES)
    bidx = jnp.arange(NBLOCKS, dtype=jnp.int32) * BLK
    block_expert = jnp.sum((bidx[:, None] >= starts[None, :]).astype(jnp.int32),
                           axis=1) - 1

    pos, v16 = pl.pallas_call(
        _kpos_body,
        grid=(1,),
        in_specs=[
            pl.BlockSpec((SEQ, LANES), lambda i: (0, 0)),
            pl.BlockSpec((SEQ, LANES), lambda i: (0, 0)),
            pl.BlockSpec((SEQ, LANES), lambda i: (0, 0)),
            pl.BlockSpec((1, LANES), lambda i: (0, 0)),
        ],
        out_specs=[
            pl.BlockSpec((SEQ, LANES), lambda i: (0, 0)),
            pl.BlockSpec((SEQ, TOPK * GW), lambda i: (0, 0)),
        ],
        out_shape=[
            jax.ShapeDtypeStruct((SEQ, LANES), jnp.int32),
            jax.ShapeDtypeStruct((SEQ, TOPK * GW), jnp.float32),
        ],
    )(tke, tkr, tkv, starts_p)

    pos7 = pos[:, :TOPK]
    pos_r = pos7.reshape(NW, _NCH, _CH)
    posk = pos7.reshape(NW, TPW, TOPK).transpose(0, 2, 1)
    v16k = v16.reshape(NW, TPW, TOPK, GW).transpose(0, 2, 1, 3)
    x3 = x2.reshape(NW, TPW, DIM)

    disp, vdisp = _sc_dispatch(x3, posk, v16k)

    eo = pl.pallas_call(
        _k4_body,
        grid_spec=pltpu.PrefetchScalarGridSpec(
            num_scalar_prefetch=1,
            grid=(NBLOCKS,),
            in_specs=[
                pl.BlockSpec((BLK, DIM), lambda i, be: (i, 0)),
                pl.BlockSpec((1, DIM, INTER), lambda i, be: (be[i], 0, 0)),
                pl.BlockSpec((1, 1, INTER), lambda i, be: (be[i], 0, 0)),
                pl.BlockSpec((1, INTER, DIM), lambda i, be: (be[i], 0, 0)),
                pl.BlockSpec((1, 1, DIM), lambda i, be: (be[i], 0, 0)),
                pl.BlockSpec((BLK, GW), lambda i, be: (i, 0)),
            ],
            out_specs=pl.BlockSpec((BLK, DIM), lambda i, be: (i, 0)),
        ),
        out_shape=jax.ShapeDtypeStruct((PAD_TOTAL, DIM), jnp.float32),
    )(block_expert, disp, w1r, b1r.reshape(NE, 1, INTER), w2r,
      b2r.reshape(NE, 1, DIM), vdisp)

    out = _sc_combine(eo, y0.reshape(SEQ // _TS, _TS, DIM), pos_r)
    return out.reshape(1, SEQ, DIM)
